# baseline (speedup 1.0000x reference)
import jax
import jax.numpy as jnp
from jax import lax
from jax.experimental import pallas as pl
from jax.experimental.pallas import tpu as pltpu
from jax.experimental.pallas import tpu_sc as plsc

_NUM_ATOMS = 320000
_FEA = 128
_NG = 10000
_NC = 2
_NS = 16
_ROW = 128
_NROWS = _NUM_ATOMS // _ROW
_COLS = _FEA // _NC
_GROWS = _NG // _NS
_BLK = _NROWS // _NS
_CH = 2
_NBUF = 4


def _pool_body(feas, owner2d, zrows, out, own_v, rows_v, *sems):
    load_sems = sems[:_NBUF]
    scat_sems = sems[_NBUF : 2 * _NBUF]
    acc = sems[2 * _NBUF]
    c = lax.axis_index("c")
    s = lax.axis_index("s")
    col0 = c * _COLS

    pltpu.sync_copy(zrows, acc.at[pl.ds(s * _GROWS, _GROWS)])
    plsc.subcore_barrier()

    base = s * _BLK + jnp.minimum(s, _NROWS % _NS)
    pltpu.sync_copy(owner2d.at[pl.ds(base, _BLK)], own_v.at[pl.ds(0, _BLK)])

    nfull = _BLK // _CH

    def feas_chunk(j):
        return feas.at[pl.ds((base + j * _CH) * _ROW, _CH * _ROW), pl.ds(col0, _COLS)]

    def fire_scatters(j, b):
        for k in range(_CH):
            pltpu.async_copy(
                rows_v.at[b, pl.ds(k * _ROW, _ROW)],
                acc.at[own_v.at[j * _CH + k]],
                scat_sems[b],
                add=True,
            )

    def drain_scatters(b):
        for _ in range(_CH):
            pltpu.make_async_copy(
                rows_v.at[b, pl.ds(0, _ROW)], acc.at[own_v.at[0]], scat_sems[b]
            ).wait()

    def step(j, bs):
        pltpu.make_async_copy(feas_chunk(j), rows_v.at[bs], load_sems[bs]).wait()
        fire_scatters(j, bs)
        br = (bs + 2) % _NBUF

        @pl.when(j >= 2)
        def _drain_prev():
            drain_scatters(br)

        @pl.when(j + 2 < nfull)
        def _refill():
            pltpu.async_copy(feas_chunk(j + 2), rows_v.at[br], load_sems[br])

    pltpu.async_copy(feas_chunk(0), rows_v.at[0], load_sems[0])
    pltpu.async_copy(feas_chunk(1), rows_v.at[1], load_sems[1])

    def quad(p, carry):
        for bs in range(_NBUF):
            step(p * _NBUF + bs, bs)
        return carry

    lax.fori_loop(0, nfull // _NBUF, quad, 0)
    for j in range(nfull - nfull % _NBUF, nfull):
        step(j, j % _NBUF)
    drain_scatters((nfull - 2) % _NBUF)
    drain_scatters((nfull - 1) % _NBUF)

    @pl.when(s < _NROWS % _NS)
    def _tail():
        pltpu.sync_copy(owner2d.at[base + _BLK], own_v.at[_BLK])
        pltpu.sync_copy(
            feas.at[pl.ds((base + _BLK) * _ROW, _ROW), pl.ds(col0, _COLS)],
            rows_v.at[0, pl.ds(0, _ROW)],
        )
        pltpu.async_copy(
            rows_v.at[0, pl.ds(0, _ROW)], acc.at[own_v.at[_BLK]], scat_sems[0], add=True
        ).wait()

    plsc.subcore_barrier()

    pltpu.sync_copy(
        acc.at[pl.ds(s * _GROWS, _GROWS)],
        out.at[pl.ds(s * _GROWS, _GROWS), pl.ds(col0, _COLS)],
    )


@jax.jit
def kernel(atom_feas, atom_owner):
    owner2d = atom_owner.astype(jnp.int32).reshape(_NROWS, _ROW)
    zrows = jnp.zeros((_GROWS, _COLS), jnp.float32)
    mesh = plsc.VectorSubcoreMesh(core_axis_name="c", subcore_axis_name="s")
    run = pl.kernel(
        _pool_body,
        out_type=jax.ShapeDtypeStruct((_NG, _FEA), jnp.float32),
        mesh=mesh,
        scratch_types=[
            pltpu.VMEM((_BLK + 1, _ROW), jnp.int32),
            pltpu.VMEM((_NBUF, _CH * _ROW, _COLS), jnp.float32),
        ]
        + [pltpu.SemaphoreType.DMA] * (2 * _NBUF)
        + [pltpu.VMEM_SHARED((_NG, _COLS), jnp.float32)],
        compiler_params=pltpu.CompilerParams(use_tc_tiling_on_sc=False),
    )
    return run(atom_feas, owner2d, zrows)


# speedup vs baseline: 1.0501x; 1.0501x over previous
import jax
import jax.numpy as jnp
from jax import lax
from jax.experimental import pallas as pl
from jax.experimental.pallas import tpu as pltpu
from jax.experimental.pallas import tpu_sc as plsc

_NUM_ATOMS = 320000
_FEA = 128
_NG = 10000
_NC = 2
_NS = 16
_ROW = 128
_NROWS = _NUM_ATOMS // _ROW
_COLS = _FEA // _NC
_GROWS = _NG // _NS
_BLK = _NROWS // _NS
_CH = 2
_NBUF = 4


def _pool_body(feas, owner2d, zrows, out, own_v, rows_v, *sems):
    load_sems = sems[:_NBUF]
    scat_sems = sems[_NBUF : 2 * _NBUF]
    acc = sems[2 * _NBUF]
    c = lax.axis_index("c")
    s = lax.axis_index("s")
    col0 = c * _COLS

    pltpu.sync_copy(zrows, acc.at[pl.ds(s * _GROWS, _GROWS)])
    plsc.subcore_barrier()

    base = s * _BLK + jnp.minimum(s, _NROWS % _NS)
    pltpu.sync_copy(owner2d.at[pl.ds(base, _BLK)], own_v.at[pl.ds(0, _BLK)])

    nfull = _BLK // _CH

    def feas_chunk(j):
        return feas.at[pl.ds((base + j * _CH) * _ROW, _CH * _ROW), pl.ds(col0, _COLS)]

    def fire_scatters(j, b):
        for k in range(_CH):
            pltpu.async_copy(
                rows_v.at[b, pl.ds(k * _ROW, _ROW)],
                acc.at[own_v.at[j * _CH + k]],
                scat_sems[b],
                add=True,
            )

    def drain_scatters(b):
        for _ in range(_CH):
            pltpu.make_async_copy(
                rows_v.at[b, pl.ds(0, _ROW)], acc.at[own_v.at[0]], scat_sems[b]
            ).wait()

    def step(j, bs):
        pltpu.make_async_copy(feas_chunk(j), rows_v.at[bs], load_sems[bs]).wait()
        fire_scatters(j, bs)
        br = (bs + 3) % _NBUF

        @pl.when(j >= 1)
        def _drain_prev():
            drain_scatters(br)

        @pl.when(j + 3 < nfull)
        def _refill():
            pltpu.async_copy(feas_chunk(j + 3), rows_v.at[br], load_sems[br])

    pltpu.async_copy(feas_chunk(0), rows_v.at[0], load_sems[0])
    pltpu.async_copy(feas_chunk(1), rows_v.at[1], load_sems[1])
    pltpu.async_copy(feas_chunk(2), rows_v.at[2], load_sems[2])

    def quad(p, carry):
        for bs in range(_NBUF):
            step(p * _NBUF + bs, bs)
        return carry

    lax.fori_loop(0, nfull // _NBUF, quad, 0)
    for j in range(nfull - nfull % _NBUF, nfull):
        step(j, j % _NBUF)
    drain_scatters((nfull - 1) % _NBUF)

    @pl.when(s < _NROWS % _NS)
    def _tail():
        pltpu.sync_copy(owner2d.at[base + _BLK], own_v.at[_BLK])
        pltpu.sync_copy(
            feas.at[pl.ds((base + _BLK) * _ROW, _ROW), pl.ds(col0, _COLS)],
            rows_v.at[0, pl.ds(0, _ROW)],
        )
        pltpu.async_copy(
            rows_v.at[0, pl.ds(0, _ROW)], acc.at[own_v.at[_BLK]], scat_sems[0], add=True
        ).wait()

    plsc.subcore_barrier()

    pltpu.sync_copy(
        acc.at[pl.ds(s * _GROWS, _GROWS)],
        out.at[pl.ds(s * _GROWS, _GROWS), pl.ds(col0, _COLS)],
    )


@jax.jit
def kernel(atom_feas, atom_owner):
    owner2d = atom_owner.astype(jnp.int32).reshape(_NROWS, _ROW)
    zrows = jnp.zeros((_GROWS, _COLS), jnp.float32)
    mesh = plsc.VectorSubcoreMesh(core_axis_name="c", subcore_axis_name="s")
    run = pl.kernel(
        _pool_body,
        out_type=jax.ShapeDtypeStruct((_NG, _FEA), jnp.float32),
        mesh=mesh,
        scratch_types=[
            pltpu.VMEM((_BLK + 1, _ROW), jnp.int32),
            pltpu.VMEM((_NBUF, _CH * _ROW, _COLS), jnp.float32),
        ]
        + [pltpu.SemaphoreType.DMA] * (2 * _NBUF)
        + [pltpu.VMEM_SHARED((_NG, _COLS), jnp.float32)],
        compiler_params=pltpu.CompilerParams(use_tc_tiling_on_sc=False),
    )
    return run(atom_feas, owner2d, zrows)
